# baseline (speedup 1.0000x reference)
import functools

import jax
import jax.numpy as jnp
from jax import lax
from jax.experimental import pallas as pl
from jax.experimental.pallas import tpu as pltpu
from jax.experimental.pallas import tpu_sc as plsc

_V = 20000
_E = 256
_H = 512
_TD = 64
_N = 128
_T = 40
_NCONV = 16
_TURNS = 8


def _sc_gather_pair(enc_tbl, dec_tbl, idx_e, idx_d):
    B = idx_e.shape[0]
    NW = 32
    per_w = B // NW
    CH = 80
    n_ch = per_w // CH
    mesh = plsc.VectorSubcoreMesh(core_axis_name="c", subcore_axis_name="s")

    @functools.partial(
        pl.kernel,
        mesh=mesh,
        out_type=(
            jax.ShapeDtypeStruct((B, _E), jnp.float32),
            jax.ShapeDtypeStruct((B, _E), jnp.float32),
        ),
        scratch_types=[
            pltpu.VMEM((CH,), jnp.int32),
            pltpu.VMEM((CH, _E), jnp.float32),
            pltpu.SemaphoreType.DMA,
        ],
    )
    def k(enc_hbm, dec_hbm, ie_hbm, id_hbm, oe_hbm, od_hbm, idx_v, rows_v, sem):
        wid = lax.axis_index("s") * 2 + lax.axis_index("c")
        base = wid * per_w
        for c in range(n_ch):
            off = base + c * CH
            pltpu.sync_copy(ie_hbm.at[pl.ds(off, CH)], idx_v)
            pltpu.async_copy(enc_hbm.at[idx_v], rows_v, sem).wait()
            pltpu.sync_copy(rows_v, oe_hbm.at[pl.ds(off, CH)])
            pltpu.sync_copy(id_hbm.at[pl.ds(off, CH)], idx_v)
            pltpu.async_copy(dec_hbm.at[idx_v], rows_v, sem).wait()
            pltpu.sync_copy(rows_v, od_hbm.at[pl.ds(off, CH)])

    return k(enc_tbl, dec_tbl, idx_e, idx_d)


def _mm_bias(A, Bm, bias, bm, bn):
    M, K = A.shape
    Nn = Bm.shape[1]
    gm = pl.cdiv(M, bm)
    gn = pl.cdiv(Nn, bn)

    def body(a_ref, b_ref, bias_ref, o_ref):
        o_ref[...] = (
            jnp.dot(a_ref[...], b_ref[...], preferred_element_type=jnp.float32)
            + bias_ref[...]
        )

    return pl.pallas_call(
        body,
        grid=(gn, gm),
        in_specs=[
            pl.BlockSpec((bm, K), lambda n, m: (m, 0)),
            pl.BlockSpec((K, bn), lambda n, m: (0, n)),
            pl.BlockSpec((1, bn), lambda n, m: (0, n)),
        ],
        out_specs=pl.BlockSpec((bm, bn), lambda n, m: (m, n)),
        out_shape=jax.ShapeDtypeStruct((M, Nn), jnp.float32),
    )(A, Bm, bias.reshape(1, -1))


def _gru_math(gi, gh, h):
    i_r, i_z, i_n = gi[:, :_H], gi[:, _H : 2 * _H], gi[:, 2 * _H :]
    h_r, h_z, h_n = gh[:, :_H], gh[:, _H : 2 * _H], gh[:, 2 * _H :]
    r = jax.nn.sigmoid(i_r + h_r)
    z = jax.nn.sigmoid(i_z + h_z)
    n = jnp.tanh(i_n + r * h_n)
    return (1.0 - z) * n + z * h


def _enc_scan(gi_seq, Wh, bh, lengths):
    def body(gi_ref, wh_ref, bh_ref, len_ref, o_ref, h_scr):
        t = pl.program_id(0)

        @pl.when(t == 0)
        def _():
            h_scr[...] = jnp.zeros_like(h_scr)

        h = h_scr[...]
        gh = (
            jnp.dot(h, wh_ref[...], preferred_element_type=jnp.float32)
            + bh_ref[...]
        )
        h_new = _gru_math(gi_ref[0], gh, h)
        h_new = jnp.where(len_ref[...] > t, h_new, h)
        h_scr[...] = h_new

        @pl.when(t == _T - 1)
        def _():
            o_ref[...] = h_new

    return pl.pallas_call(
        body,
        grid=(_T,),
        in_specs=[
            pl.BlockSpec((1, _N, 3 * _H), lambda t: (t, 0, 0)),
            pl.BlockSpec((_H, 3 * _H), lambda t: (0, 0)),
            pl.BlockSpec((1, 3 * _H), lambda t: (0, 0)),
            pl.BlockSpec((_N, 1), lambda t: (0, 0)),
        ],
        out_specs=pl.BlockSpec((_N, _H), lambda t: (0, 0)),
        out_shape=jax.ShapeDtypeStruct((_N, _H), jnp.float32),
        scratch_shapes=[pltpu.VMEM((_N, _H), jnp.float32)],
    )(gi_seq, Wh, bh.reshape(1, -1), lengths)


def _mid(enc_h, W_topic, b_topic, ctx_Wx, ctx_Wh, ctx_bx, ctx_bh, c2d_W,
         c2d_b, dec_Wx_bot, turns):
    def body(eh_ref, eh3_ref, wt_ref, bt_ref, cwx_ref, cwh_ref, cbx_ref,
             cbh_ref, cw_ref, cb_ref, dwb_ref, tn_ref, dinit_ref, gsh_ref,
             ctx_scr):
        eh = eh_ref[...]
        topic = jnp.tanh(
            jnp.dot(eh, wt_ref[...], preferred_element_type=jnp.float32)
            + bt_ref[...]
        )
        gsh_ref[...] = jnp.dot(
            topic, dwb_ref[...], preferred_element_type=jnp.float32
        )
        h = jnp.zeros((_NCONV, _H), jnp.float32)
        for t in range(_TURNS):
            x_t = eh3_ref[:, t, :]
            gi = (
                jnp.dot(x_t, cwx_ref[...], preferred_element_type=jnp.float32)
                + cbx_ref[...]
            )
            gh = (
                jnp.dot(h, cwh_ref[...], preferred_element_type=jnp.float32)
                + cbh_ref[...]
            )
            h_new = _gru_math(gi, gh, h)
            h = jnp.where(tn_ref[...] > t, h_new, h)
            ctx_scr[:, t, :] = h
        ctxf = ctx_scr[...].reshape(_N, _H)
        dinit_ref[...] = (
            jnp.dot(ctxf, cw_ref[...], preferred_element_type=jnp.float32)
            + cb_ref[...]
        )

    return pl.pallas_call(
        body,
        out_shape=(
            jax.ShapeDtypeStruct((_N, _H), jnp.float32),
            jax.ShapeDtypeStruct((_N, 3 * _H), jnp.float32),
        ),
        scratch_shapes=[pltpu.VMEM((_NCONV, _TURNS, _H), jnp.float32)],
    )(
        enc_h,
        enc_h.reshape(_NCONV, _TURNS, _H),
        W_topic,
        b_topic.reshape(1, -1),
        ctx_Wx,
        ctx_Wh,
        ctx_bx.reshape(1, -1),
        ctx_bh.reshape(1, -1),
        c2d_W,
        c2d_b.reshape(1, -1),
        dec_Wx_bot,
        turns,
    )


def _dec_scan(gi_seq, gshift, Wh, bh, h0):
    def body(gi_ref, gs_ref, wh_ref, bh_ref, h0_ref, o_ref, h_scr):
        t = pl.program_id(0)

        @pl.when(t == 0)
        def _():
            h_scr[...] = h0_ref[...]

        h = h_scr[...]
        gi = gi_ref[0] + gs_ref[...]
        gh = (
            jnp.dot(h, wh_ref[...], preferred_element_type=jnp.float32)
            + bh_ref[...]
        )
        h_new = _gru_math(gi, gh, h)
        h_scr[...] = h_new
        o_ref[0] = h_new

    return pl.pallas_call(
        body,
        grid=(_T,),
        in_specs=[
            pl.BlockSpec((1, _N, 3 * _H), lambda t: (t, 0, 0)),
            pl.BlockSpec((_N, 3 * _H), lambda t: (0, 0)),
            pl.BlockSpec((_H, 3 * _H), lambda t: (0, 0)),
            pl.BlockSpec((1, 3 * _H), lambda t: (0, 0)),
            pl.BlockSpec((_N, _H), lambda t: (0, 0)),
        ],
        out_specs=pl.BlockSpec((1, _N, _H), lambda t: (t, 0, 0)),
        out_shape=jax.ShapeDtypeStruct((_T, _N, _H), jnp.float32),
        scratch_shapes=[pltpu.VMEM((_N, _H), jnp.float32)],
    )(gi_seq, gshift, Wh, bh.reshape(1, -1), h0)


def _to_batch_major(dhs):
    def body(x_ref, o_ref):
        for j in range(8):
            o_ref[j] = x_ref[:, j, :]

    return pl.pallas_call(
        body,
        grid=(_N // 8,),
        in_specs=[pl.BlockSpec((_T, 8, _H), lambda n: (0, n, 0))],
        out_specs=pl.BlockSpec((8, _T, _H), lambda n: (n, 0, 0)),
        out_shape=jax.ShapeDtypeStruct((_N, _T, _H), jnp.float32),
    )(dhs)


def kernel(enc_embed, enc_Wx, enc_Wh, enc_bx, enc_bh, W_topic, b_topic,
           ctx_Wx, ctx_Wh, ctx_bx, ctx_bh, c2d_W, c2d_b, dec_embed, dec_Wx,
           dec_Wh, dec_bx, dec_bh, out_W, out_b, input_sentences,
           input_lengths, input_turns, target_sentences):
    idx_e = input_sentences.astype(jnp.int32).T.reshape(-1)
    idx_d = target_sentences.astype(jnp.int32).T.reshape(-1)
    x_enc, t_emb = _sc_gather_pair(enc_embed, dec_embed, idx_e, idx_d)

    gi_enc = _mm_bias(x_enc, enc_Wx, enc_bx, 512, 3 * _H)
    enc_h = _enc_scan(
        gi_enc.reshape(_T, _N, 3 * _H),
        enc_Wh,
        enc_bh,
        input_lengths.astype(jnp.int32).reshape(_N, 1),
    )

    dec_init, gshift = _mid(
        enc_h, W_topic, b_topic, ctx_Wx, ctx_Wh, ctx_bx, ctx_bh, c2d_W,
        c2d_b, dec_Wx[_E:], input_turns.astype(jnp.int32).reshape(_NCONV, 1),
    )

    gi_dec = _mm_bias(t_emb, dec_Wx[:_E], dec_bx, 512, 3 * _H)
    dhs_t = _dec_scan(
        gi_dec.reshape(_T, _N, 3 * _H), gshift, dec_Wh, dec_bh, dec_init
    )
    dhs = _to_batch_major(dhs_t)

    logits = _mm_bias(dhs.reshape(_N * _T, _H), out_W, out_b, 512, 2048)
    return logits.reshape(_N, _T, _V)
